# baseline (speedup 1.0000x reference)
import functools

import jax
import jax.numpy as jnp
from jax import lax
from jax.experimental import pallas as pl
from jax.experimental.pallas import tpu as pltpu
from jax.experimental.pallas import tpu_sc as plsc

B, T, N, F_IN = 256, 8, 30, 128
NF, C = 64, 10
NC, NS = 2, 16
NW = NC * NS
GPW = B // NW
L = 16

NN = N * N
SIG = T * NN
XWF = N * NF
MF = C * N * NF
NEG = -1e30


GX = 32


def _xw_body(real_ref, wconv_ref, out_ref):
    x = real_ref[...].reshape(GX * N, F_IN)
    xw = jnp.dot(x, wconv_ref[...], preferred_element_type=jnp.float32,
                 precision=lax.Precision.HIGHEST)
    out_ref[...] = (xw * (1.0 / T)).reshape(GX, N, NF)


def _tc_xw(real, W_conv):
    return pl.pallas_call(
        _xw_body,
        grid=(B // GX,),
        in_specs=[pl.BlockSpec((GX, N, F_IN), lambda i: (i, 0, 0)),
                  pl.BlockSpec((F_IN, NF), lambda i: (0, 0))],
        out_specs=pl.BlockSpec((GX, N, NF), lambda i: (i, 0, 0)),
        out_shape=jax.ShapeDtypeStruct((B, N, NF), jnp.float32),
    )(real, W_conv)


_JBLOCKS = ((0, 8), (8, 8), (16, 8), (24, 6))


def _bfly(v, op):
    dnums = lax.GatherDimensionNumbers(offset_dims=(), collapsed_slice_dims=(0,),
                                       start_index_map=(0,))
    lane = lax.iota(jnp.int32, L)
    for s in (8, 4, 2, 1):
        perm = (lane ^ s)[:, None]
        g = lax.gather(v, perm, dnums, (1,),
                       mode=lax.GatherScatterMode.PROMISE_IN_BOUNDS)
        v = op(v, g)
    return v


def _sc_body(sigs_hbm, xw_hbm, m_hbm, bconv_hbm, bias2_hbm, out_hbm,
             sigs_v, xw_v, m_v, bconv_v, bias2_v, out_v, adj_v, agg_v):
    cid = lax.axis_index("c")
    sid = lax.axis_index("s")
    wid = sid * NC + cid

    pltpu.sync_copy(m_hbm, m_v)
    pltpu.sync_copy(bconv_hbm, bconv_v)
    pltpu.sync_copy(bias2_hbm, bias2_v)
    bconv_r = [bconv_v[pl.ds(fv * L, L)] for fv in range(NF // L)]
    bias2_r = bias2_v[...]

    def graph_body(k, _):
        g = wid * GPW + k
        pltpu.sync_copy(sigs_hbm.at[g], sigs_v)
        pltpu.sync_copy(xw_hbm.at[g], xw_v)

        def mean_body(c2, carry):
            base = jnp.minimum(c2 * L, NN - L)
            acc = sigs_v[pl.ds(base, L)]
            for t in range(1, T):
                acc = acc + sigs_v[pl.ds(t * NN + base, L)]
            adj_v[pl.ds(base, L)] = acc
            return carry

        lax.fori_loop(0, (NN + L - 1) // L, mean_body, 0)

        for (j0, J) in _JBLOCKS:
            def agg_body(i, accs, j0=j0, J=J):
                xwr = [xw_v[pl.ds(i * NF + fv * L, L)] for fv in range(NF // L)]
                av = adj_v[pl.ds(i * N + j0, L)]
                out = []
                for jj in range(J):
                    a = av[jj]
                    row = accs[jj]
                    out.append(tuple(row[fv] + a * xwr[fv]
                                     for fv in range(NF // L)))
                return tuple(out)

            init = tuple(tuple(jnp.zeros((L,), jnp.float32)
                               for _ in range(NF // L)) for _ in range(J))
            accs = lax.fori_loop(0, N, agg_body, init)
            for jj in range(J):
                for fv in range(NF // L):
                    agg_v[pl.ds((j0 + jj) * NF + fv * L, L)] = accs[jj][fv]

        def head_body(j, accC):
            jb = j * NF
            out = list(accC)
            for fv in range(NF // L):
                h = jnp.maximum(agg_v[pl.ds(jb + fv * L, L)] + bconv_r[fv], 0.0)
                for c in range(C):
                    out[c] = out[c] + m_v[pl.ds(c * XWF + jb + fv * L, L)] * h
            return tuple(out)

        accC = lax.fori_loop(0, N, head_body,
                             tuple(jnp.zeros((L,), jnp.float32)
                                   for _ in range(C)))
        lane = lax.iota(jnp.int32, L)
        zero = jnp.zeros((L,), jnp.float32)
        lv = bias2_r
        for c in range(C):
            lv = lv + jnp.where(lane == c, _bfly(accC[c], jnp.add), zero)
        mxv = _bfly(lv, jnp.maximum)
        e = jnp.exp(lv - mxv)
        sv = _bfly(e, jnp.add)
        out_v[pl.ds(k * L, L)] = e / sv
        return 0

    lax.fori_loop(0, GPW, graph_body, 0)
    pltpu.sync_copy(out_v, out_hbm.at[pl.ds(wid * GPW * L, GPW * L)])


@functools.lru_cache(maxsize=1)
def _sc_main():
    return pl.kernel(
        _sc_body,
        out_type=jax.ShapeDtypeStruct((B * L,), jnp.float32),
        mesh=plsc.VectorSubcoreMesh(core_axis_name="c", subcore_axis_name="s",
                                    num_cores=NC, num_subcores=NS),
        scratch_types=[
            pltpu.VMEM((SIG,), jnp.float32),
            pltpu.VMEM((XWF,), jnp.float32),
            pltpu.VMEM((MF,), jnp.float32),
            pltpu.VMEM((NF,), jnp.float32),
            pltpu.VMEM((L,), jnp.float32),
            pltpu.VMEM((GPW * L,), jnp.float32),
            pltpu.VMEM((NN + L,), jnp.float32),
            pltpu.VMEM((XWF,), jnp.float32),
        ],
    )


def kernel(real, imag, graph_sigs, W_conv, b_conv, w_pool, b_pool, W_head, b_head):
    del imag
    xw = _tc_xw(real, W_conv).reshape(B, XWF)
    sigs = graph_sigs.reshape(B, SIG)
    whead = W_head.reshape(C, N)
    m = (whead[:, :, None] * w_pool[:, 0][None, None, :]).reshape(MF)
    bias2 = b_head + b_pool[0] * jnp.sum(whead, axis=1)
    bias2_p = jnp.concatenate([bias2, jnp.full((L - C,), NEG, jnp.float32)])
    out = _sc_main()(sigs, xw, m, b_conv, bias2_p)
    return out.reshape(B, L)[:, :C]
